# Initial kernel scaffold; baseline (speedup 1.0000x reference)
import functools

import jax
import jax.numpy as jnp
from jax import lax
from jax.experimental import pallas as pl
from jax.experimental.pallas import tpu as pltpu
from jax.experimental.pallas import tpu_sc as plsc

F32 = jnp.float32
I32 = jnp.int32

NC = 2
NS = 16
NW = NC * NS
K = 80


def _proj_body(x_r, ea_r, w_r, a2_r, mb_r, h_r, asd_r, ae_r):
    xb = x_r[...]
    h_r[...] = jnp.dot(xb, w_r[...], preferred_element_type=F32)
    asd_r[...] = jnp.dot(xb, a2_r[...], preferred_element_type=F32)
    ae_r[...] = jnp.dot(ea_r[...], mb_r[...], preferred_element_type=F32)


def _tc_proj(x, ea2, W, A2, Mbig):
    n, d = x.shape
    er, ek = ea2.shape
    bp = 1000
    grid = (n // bp,)
    return pl.pallas_call(
        _proj_body,
        grid=grid,
        in_specs=[
            pl.BlockSpec((bp, d), lambda i: (i, 0)),
            pl.BlockSpec((bp, ek), lambda i: (i, 0)),
            pl.BlockSpec((d, 128), lambda i: (0, 0)),
            pl.BlockSpec((d, 8), lambda i: (0, 0)),
            pl.BlockSpec((ek, 128), lambda i: (0, 0)),
        ],
        out_specs=[
            pl.BlockSpec((bp, 128), lambda i: (i, 0)),
            pl.BlockSpec((bp, 8), lambda i: (i, 0)),
            pl.BlockSpec((bp, 128), lambda i: (i, 0)),
        ],
        out_shape=[
            jax.ShapeDtypeStruct((n, 128), F32),
            jax.ShapeDtypeStruct((n, 8), F32),
            jax.ShapeDtypeStruct((er, 128), F32),
        ],
    )(x, ea2, W, A2, Mbig)


def _make_sc_kernel(n_nodes, n_edges):
    epw = n_edges // NW
    nch = epw // K
    assert nch * K == epw
    rps = n_nodes // NS
    mesh = plsc.VectorSubcoreMesh(
        core_axis_name="c", subcore_axis_name="s",
        num_cores=NC, num_subcores=NS)

    @functools.partial(
        pl.kernel,
        out_type=(
            jax.ShapeDtypeStruct((NC, n_nodes, 128), F32),
            jax.ShapeDtypeStruct((NC, n_nodes, 8), F32),
        ),
        mesh=mesh,
        scratch_types=(
            pltpu.VMEM_SHARED((n_nodes, 128), F32),
            pltpu.VMEM_SHARED((n_nodes, 8), F32),
            pltpu.VMEM((n_nodes * 8,), F32),
            pltpu.VMEM((K,), I32),
            pltpu.VMEM((K,), I32),
            pltpu.VMEM((K * 4,), F32),
            pltpu.VMEM((K, 128), F32),
            pltpu.VMEM((K, 8), F32),
            pltpu.VMEM((4 * K,), F32),
            pltpu.SemaphoreType.DMA,
        ),
    )
    def sc_gat(h_hbm, asd_hbm, ae_hbm, ei_hbm, z_hbm, zd_hbm,
               raw_out, den_out,
               raw_sp, den_sp, ast_v, src_v, dst_v, ae_v, rows_v,
               srows_v, st_v, sem):
        cid = lax.axis_index("c")
        sid = lax.axis_index("s")
        wid = sid * NC + cid
        iota = lax.iota(I32, 16)
        zv = jnp.zeros((16,), F32)

        pltpu.sync_copy(z_hbm.at[pl.ds(sid * rps, rps)],
                        raw_sp.at[pl.ds(sid * rps, rps)])
        pltpu.sync_copy(zd_hbm.at[pl.ds(sid * rps, rps)],
                        den_sp.at[pl.ds(sid * rps, rps)])
        pltpu.sync_copy(asd_hbm, ast_v)
        for eb in range(K // 16):
            rowi = iota + eb * 16
            for hh in range(4, 8):
                plsc.store_scatter(
                    srows_v, [rowi, jnp.full((16,), hh, I32)], zv)
        plsc.subcore_barrier()

        base = wid * epw

        def do_chunk(e0):
            pltpu.sync_copy(ei_hbm.at[0, pl.ds(e0, K)], src_v)
            pltpu.sync_copy(ei_hbm.at[1, pl.ds(e0, K)], dst_v)
            pltpu.sync_copy(ae_hbm.at[pl.ds(e0 * 4, K * 4)], ae_v)
            cp = pltpu.async_copy(h_hbm.at[src_v], rows_v, sem)
            for eb in range(K // 16):
                sidx = src_v[pl.ds(eb * 16, 16)]
                didx = dst_v[pl.ds(eb * 16, 16)]
                rowi = iota + eb * 16
                for hh in range(4):
                    va = plsc.load_gather(ast_v, [sidx * 8 + hh])
                    vb = plsc.load_gather(ast_v, [didx * 8 + (4 + hh)])
                    ve = plsc.load_gather(ae_v, [rowi * 4 + hh])
                    al = va + vb + ve
                    al = jnp.where(al >= 0.0, al, al * 0.2)
                    sv = jnp.exp(al)
                    plsc.store_scatter(
                        srows_v, [rowi, jnp.full((16,), hh, I32)], sv)
                    st_v[pl.ds(hh * K + eb * 16, 16)] = sv
            cp.wait()
            pltpu.sync_copy(srows_v, den_sp.at[dst_v], add=True)
            for eb in range(K // 16):
                rowi = iota + eb * 16
                for hh in range(4):
                    sv = st_v[pl.ds(hh * K + eb * 16, 16)]
                    for q in range(2):
                        colv = jnp.full((16,), hh * 32 + q * 16, I32)
                        vr = plsc.load_gather(rows_v, [rowi, colv])
                        plsc.store_scatter(rows_v, [rowi, colv], vr * sv)
            pltpu.sync_copy(rows_v, raw_sp.at[dst_v], add=True)

        def chunk_body(ci, carry):
            do_chunk(base + ci * K)
            return carry

        lax.fori_loop(0, nch, chunk_body, 0)

        plsc.subcore_barrier()
        pltpu.sync_copy(raw_sp.at[pl.ds(sid * rps, rps)],
                        raw_out.at[cid, pl.ds(sid * rps, rps)])
        pltpu.sync_copy(den_sp.at[pl.ds(sid * rps, rps)],
                        den_out.at[cid, pl.ds(sid * rps, rps)])

    return sc_gat


def _finalize_body(x_r, raw_r, den_r, rmat_r, b_r, g_r, be_r, out_r):
    agg = raw_r[0] + raw_r[1]
    den = den_r[0] + den_r[1]
    dexp = jnp.dot(den, rmat_r[...], preferred_element_type=F32)
    o = x_r[...] + agg / (dexp + 1e-16) + b_r[...]
    mu = jnp.mean(o, axis=-1, keepdims=True)
    d = o - mu
    var = jnp.mean(d * d, axis=-1, keepdims=True)
    out_r[...] = d * lax.rsqrt(var + 1e-5) * g_r[...] + be_r[...]


def _tc_finalize(x, raw, den, rmat, bias, gamma, beta):
    n, dd = x.shape
    bp = 1000
    grid = (n // bp,)
    return pl.pallas_call(
        _finalize_body,
        grid=grid,
        in_specs=[
            pl.BlockSpec((bp, dd), lambda i: (i, 0)),
            pl.BlockSpec((NC, bp, 128), lambda i: (0, i, 0)),
            pl.BlockSpec((NC, bp, 8), lambda i: (0, i, 0)),
            pl.BlockSpec((8, 128), lambda i: (0, 0)),
            pl.BlockSpec((1, 128), lambda i: (0, 0)),
            pl.BlockSpec((1, 128), lambda i: (0, 0)),
            pl.BlockSpec((1, 128), lambda i: (0, 0)),
        ],
        out_specs=pl.BlockSpec((bp, dd), lambda i: (i, 0)),
        out_shape=jax.ShapeDtypeStruct((n, dd), F32),
    )(x, raw, den, rmat, bias, gamma, beta)


def kernel(x, edge_index, edge_attr, W, att_src, att_dst, W_edge, att_edge,
           bias, ln_gamma, ln_beta):
    n, d = x.shape
    e = edge_index.shape[1]
    h_heads, c = att_src.shape
    ed = edge_attr.shape[1]

    ei = edge_index.astype(I32)

    A_src = (W.reshape(d, h_heads, c) * att_src[None]).sum(-1)
    A_dst = (W.reshape(d, h_heads, c) * att_dst[None]).sum(-1)
    A2 = jnp.concatenate([A_src, A_dst], axis=1)
    M4 = (W_edge.reshape(ed, h_heads, c) * att_edge[None]).sum(-1)
    Mbig = jnp.kron(jnp.eye(32, dtype=F32), M4)
    ea2 = edge_attr.reshape(e // 32, ed * 32)

    h, asd, aef = _tc_proj(x, ea2, W, A2, Mbig)
    asd_flat = asd.reshape(-1)
    ae_flat = aef.reshape(-1)

    zeros = jnp.zeros((n, 128), F32)
    zerosd = jnp.zeros((n, 8), F32)
    sc = _make_sc_kernel(n, e)
    raw, den = sc(h, asd_flat, ae_flat, ei, zeros, zerosd)

    rmat = jnp.concatenate(
        [jnp.kron(jnp.eye(4, dtype=F32), jnp.ones((1, 32), F32)),
         jnp.zeros((4, 128), F32)], axis=0)
    out = _tc_finalize(x, raw, den, rmat,
                       bias.reshape(1, 128), ln_gamma.reshape(1, 128),
                       ln_beta.reshape(1, 128))
    return out


# speedup vs baseline: 36.0175x; 36.0175x over previous
import functools

import jax
import jax.numpy as jnp
from jax import lax
from jax.experimental import pallas as pl
from jax.experimental.pallas import tpu as pltpu
from jax.experimental.pallas import tpu_sc as plsc

F32 = jnp.float32
I32 = jnp.int32

NC = 2
NS = 16
NW = NC * NS
K = 80


def _proj_body(x_r, ea_r, w_r, as_r, ad_r, mb_r, h_r, asrc_r, adst_r, ae_r):
    xb = x_r[...]
    h_r[...] = jnp.dot(xb, w_r[...], preferred_element_type=F32)
    asrc_r[...] = jnp.dot(xb, as_r[...], preferred_element_type=F32)
    adst_r[...] = jnp.dot(xb, ad_r[...], preferred_element_type=F32)
    ae_r[...] = jnp.dot(ea_r[...], mb_r[...], preferred_element_type=F32)


def _tc_proj(x, ea2, W, A_src16, A_dst16, Mbig):
    n, d = x.shape
    er, ek = ea2.shape
    em = Mbig.shape[1]
    bp = 1000
    grid = (n // bp,)
    return pl.pallas_call(
        _proj_body,
        grid=grid,
        in_specs=[
            pl.BlockSpec((bp, d), lambda i: (i, 0)),
            pl.BlockSpec((bp, ek), lambda i: (i, 0)),
            pl.BlockSpec((d, 128), lambda i: (0, 0)),
            pl.BlockSpec((d, 16), lambda i: (0, 0)),
            pl.BlockSpec((d, 16), lambda i: (0, 0)),
            pl.BlockSpec((ek, em), lambda i: (0, 0)),
        ],
        out_specs=[
            pl.BlockSpec((bp, 128), lambda i: (i, 0)),
            pl.BlockSpec((bp, 16), lambda i: (i, 0)),
            pl.BlockSpec((bp, 16), lambda i: (i, 0)),
            pl.BlockSpec((bp, em), lambda i: (i, 0)),
        ],
        out_shape=[
            jax.ShapeDtypeStruct((n, 128), F32),
            jax.ShapeDtypeStruct((n, 16), F32),
            jax.ShapeDtypeStruct((n, 16), F32),
            jax.ShapeDtypeStruct((er, em), F32),
        ],
    )(x, ea2, W, A_src16, A_dst16, Mbig)


def _make_sc_kernel(n_nodes, n_pad, n_edges):
    epw = n_edges // NW
    nch = epw // K
    assert nch * K == epw
    rps = n_pad // NS
    assert rps % 8 == 0
    mesh = plsc.VectorSubcoreMesh(
        core_axis_name="c", subcore_axis_name="s",
        num_cores=NC, num_subcores=NS)

    @functools.partial(
        pl.kernel,
        out_type=(
            jax.ShapeDtypeStruct((NC, n_pad, 128), F32),
            jax.ShapeDtypeStruct((NC, n_pad, 16), F32),
        ),
        mesh=mesh,
        compiler_params=pltpu.CompilerParams(use_tc_tiling_on_sc=False),
        scratch_types=(
            pltpu.VMEM_SHARED((n_pad, 128), F32),
            pltpu.VMEM_SHARED((n_pad, 16), F32),
            pltpu.VMEM((K,), I32),
            pltpu.VMEM((K,), I32),
            pltpu.VMEM((K, 16), F32),
            pltpu.VMEM((K, 16), F32),
            pltpu.VMEM((K, 16), F32),
            pltpu.VMEM((K, 16), F32),
            pltpu.VMEM((K, 128), F32),
            pltpu.SemaphoreType.DMA,
            pltpu.SemaphoreType.DMA,
            pltpu.SemaphoreType.DMA,
        ),
    )
    def sc_gat(h_hbm, asrc_hbm, adst_hbm, ae_hbm, src_hbm, dst_hbm,
               z_hbm, zd_hbm,
               raw_out, den_out,
               raw_sp, den_sp, src_v, dst_v, asrc_v, adst_v, ae_v,
               srows_v, rows_v, sem0, sem1, sem2):
        cid = lax.axis_index("c")
        sid = lax.axis_index("s")
        wid = sid * NC + cid

        pltpu.sync_copy(z_hbm.at[pl.ds(sid * rps, rps)],
                        raw_sp.at[pl.ds(sid * rps, rps)])
        pltpu.sync_copy(zd_hbm.at[pl.ds(sid * rps, rps)],
                        den_sp.at[pl.ds(sid * rps, rps)])
        plsc.subcore_barrier()

        base = wid * epw

        def do_chunk(e0):
            pltpu.sync_copy(src_hbm.at[pl.ds(e0, K)], src_v)
            pltpu.sync_copy(dst_hbm.at[pl.ds(e0, K)], dst_v)
            cph = pltpu.async_copy(h_hbm.at[src_v], rows_v, sem0)
            cpa = pltpu.async_copy(asrc_hbm.at[src_v], asrc_v, sem1)
            cpb = pltpu.async_copy(adst_hbm.at[dst_v], adst_v, sem2)
            pltpu.sync_copy(ae_hbm.at[pl.ds(e0, K)], ae_v)
            cpa.wait()
            cpb.wait()

            def alpha_body(e, c):
                al = asrc_v[e, :] + adst_v[e, :] + ae_v[e, :]
                al = jnp.where(al >= 0.0, al, al * 0.2)
                srows_v[e, :] = jnp.exp(al)
                return c

            lax.fori_loop(0, K, alpha_body, 0)
            pltpu.sync_copy(srows_v, den_sp.at[dst_v], add=True)
            cph.wait()

            gdn = lax.GatherDimensionNumbers(
                offset_dims=(), collapsed_slice_dims=(0,),
                start_index_map=(0,))

            def mul_body(e, c):
                s16 = srows_v[e, :]
                for hh in range(4):
                    sv = lax.gather(
                        s16, jnp.full((16, 1), hh, I32), gdn,
                        slice_sizes=(1,),
                        mode=lax.GatherScatterMode.PROMISE_IN_BOUNDS)
                    for q in range(2):
                        off = hh * 32 + q * 16
                        rows_v[e, pl.ds(off, 16)] = (
                            rows_v[e, pl.ds(off, 16)] * sv)
                return c

            lax.fori_loop(0, K, mul_body, 0)
            pltpu.sync_copy(rows_v, raw_sp.at[dst_v], add=True)

        def chunk_body(ci, carry):
            do_chunk(base + ci * K)
            return carry

        lax.fori_loop(0, nch, chunk_body, 0)

        plsc.subcore_barrier()
        pltpu.sync_copy(raw_sp.at[pl.ds(sid * rps, rps)],
                        raw_out.at[cid, pl.ds(sid * rps, rps)])
        pltpu.sync_copy(den_sp.at[pl.ds(sid * rps, rps)],
                        den_out.at[cid, pl.ds(sid * rps, rps)])

    return sc_gat


def _finalize_body(x_r, raw_r, den_r, rmat_r, b_r, g_r, be_r, out_r):
    agg = raw_r[0] + raw_r[1]
    den = den_r[0] + den_r[1]
    dexp = jnp.dot(den, rmat_r[...], preferred_element_type=F32)
    o = x_r[...] + agg / (dexp + 1e-16) + b_r[...]
    mu = jnp.mean(o, axis=-1, keepdims=True)
    d = o - mu
    var = jnp.mean(d * d, axis=-1, keepdims=True)
    out_r[...] = d * lax.rsqrt(var + 1e-5) * g_r[...] + be_r[...]


def _tc_finalize(x, raw, den, rmat, bias, gamma, beta):
    n, dd = x.shape
    bp = 1000
    grid = (n // bp,)
    return pl.pallas_call(
        _finalize_body,
        grid=grid,
        in_specs=[
            pl.BlockSpec((bp, dd), lambda i: (i, 0)),
            pl.BlockSpec((NC, bp, 128), lambda i: (0, i, 0)),
            pl.BlockSpec((NC, bp, 16), lambda i: (0, i, 0)),
            pl.BlockSpec((16, 128), lambda i: (0, 0)),
            pl.BlockSpec((1, 128), lambda i: (0, 0)),
            pl.BlockSpec((1, 128), lambda i: (0, 0)),
            pl.BlockSpec((1, 128), lambda i: (0, 0)),
        ],
        out_specs=pl.BlockSpec((bp, dd), lambda i: (i, 0)),
        out_shape=jax.ShapeDtypeStruct((n, dd), F32),
    )(x, raw, den, rmat, bias, gamma, beta)


def kernel(x, edge_index, edge_attr, W, att_src, att_dst, W_edge, att_edge,
           bias, ln_gamma, ln_beta):
    n, d = x.shape
    e = edge_index.shape[1]
    h_heads, c = att_src.shape
    ed = edge_attr.shape[1]

    ei = edge_index.astype(I32)

    A_src = (W.reshape(d, h_heads, c) * att_src[None]).sum(-1)
    A_dst = (W.reshape(d, h_heads, c) * att_dst[None]).sum(-1)
    pad12 = jnp.zeros((d, 16 - h_heads), F32)
    A_src16 = jnp.concatenate([A_src, pad12], axis=1)
    A_dst16 = jnp.concatenate([A_dst, pad12], axis=1)
    M4 = (W_edge.reshape(ed, h_heads, c) * att_edge[None]).sum(-1)
    M16 = jnp.concatenate(
        [M4, jnp.zeros((ed, 16 - h_heads), F32)], axis=1)
    Mbig = jnp.kron(jnp.eye(32, dtype=F32), M16)
    ea2 = edge_attr.reshape(e // 32, ed * 32)

    h, asrc16, adst16, aef = _tc_proj(x, ea2, W, A_src16, A_dst16, Mbig)
    ae16 = aef.reshape(e, 16)

    n_pad = ((n + NS * 8 - 1) // (NS * 8)) * NS * 8
    zeros = jnp.zeros((n_pad, 128), F32)
    zerosd = jnp.zeros((n_pad, 16), F32)
    sc = _make_sc_kernel(n, n_pad, e)
    raw, den = sc(h, asrc16, adst16, ae16, ei[0], ei[1], zeros, zerosd)

    rmat = jnp.concatenate(
        [jnp.kron(jnp.eye(4, dtype=F32), jnp.ones((1, 32), F32)),
         jnp.zeros((12, 128), F32)], axis=0)
    out = _tc_finalize(x, raw, den, rmat,
                       bias.reshape(1, 128), ln_gamma.reshape(1, 128),
                       ln_beta.reshape(1, 128))
    return out
